# baseline (speedup 1.0000x reference)
import jax
import jax.numpy as jnp
from jax.experimental import pallas as pl
from jax.experimental.pallas import tpu as pltpu

NUM_NODES = 200


def _mlp_head_pallas(x, W1, b1, W2, b2, W3, b3):
    B, SIN = x.shape
    S1 = W1.shape[1]
    S2 = W2.shape[1]
    OUT = W3.shape[1]
    BLK = 640
    nb = S1 // BLK

    def body(x_ref, w1_ref, b1_ref, w2_ref, b2_ref, w3_ref, b3_ref, out_ref,
             h1_ref):
        j = pl.program_id(0)
        h1_ref[:, pl.ds(j * BLK, BLK)] = jnp.maximum(
            x_ref[:, :] @ w1_ref[:, :] + b1_ref[0, :], 0.0)

        @pl.when(j == nb - 1)
        def _():
            h2 = jnp.maximum(h1_ref[:, :] @ w2_ref[:, :] + b2_ref[0, :], 0.0)
            logits = h2 @ w3_ref[:, :] + b3_ref[0, :]
            m = jnp.max(logits, axis=-1, keepdims=True)
            e = jnp.exp(logits - m)
            out_ref[:, :] = e / jnp.sum(e, axis=-1, keepdims=True)

    return pl.pallas_call(
        body,
        grid=(nb,),
        in_specs=[
            pl.BlockSpec((B, SIN), lambda j: (0, 0)),
            pl.BlockSpec((SIN, BLK), lambda j: (0, j)),
            pl.BlockSpec((1, BLK), lambda j: (0, j)),
            pl.BlockSpec((S1, S2), lambda j: (0, 0)),
            pl.BlockSpec((1, S2), lambda j: (0, 0)),
            pl.BlockSpec((S2, OUT), lambda j: (0, 0)),
            pl.BlockSpec((1, OUT), lambda j: (0, 0)),
        ],
        out_specs=pl.BlockSpec((B, OUT), lambda j: (0, 0)),
        out_shape=jax.ShapeDtypeStruct((B, OUT), jnp.float32),
        scratch_shapes=[pltpu.VMEM((B, S1), jnp.float32)],
    )(x, W1, b1.reshape(1, -1), W2, b2.reshape(1, -1), W3, b3.reshape(1, -1))


def kernel(x_temporal, edge_index, edge_weight, batch, W_xz, b_xz, W_hz, b_hz,
           W_xr, b_xr, W_hr, b_hr, W_xh, b_xh, W_hh, b_hh, W1, b1, W2, b2,
           W3, b3):
    n = x_temporal.shape[1]
    src, dst = edge_index[0], edge_index[1]
    deg = jnp.zeros((n,), x_temporal.dtype).at[dst].add(edge_weight)
    safe = jnp.where(deg > 0, deg, 1.0)
    dis = jnp.where(deg > 0, 1.0 / jnp.sqrt(safe), 0.0)
    norm = -dis[src] * edge_weight * dis[dst]

    def prop(x):
        msg = x[src] * norm[:, None]
        return jnp.zeros_like(x).at[dst].add(msg)

    def cheb(x, W, b):
        Tx0 = x
        out = Tx0 @ W[0]
        Tx1 = prop(Tx0)
        out = out + Tx1 @ W[1]
        for k in range(2, W.shape[0]):
            Tx2 = 2.0 * prop(Tx1) - Tx0
            out = out + Tx2 @ W[k]
            Tx0, Tx1 = Tx1, Tx2
        return out + b

    H = jnp.zeros((n, x_temporal.shape[2]), x_temporal.dtype)
    for t in range(x_temporal.shape[0]):
        X = x_temporal[t]
        Z = jax.nn.sigmoid(cheb(X, W_xz, b_xz) + cheb(H, W_hz, b_hz))
        R = jax.nn.sigmoid(cheb(X, W_xr, b_xr) + cheb(H, W_hr, b_hr))
        Htil = jnp.tanh(cheb(X, W_xh, b_xh) + cheb(R * H, W_hh, b_hh))
        H = Z * H + (1.0 - Z) * Htil
        H = jax.nn.relu(H)
    bsz = -(-batch.shape[0] // NUM_NODES)
    x = H.reshape(bsz, NUM_NODES * x_temporal.shape[2])
    return _mlp_head_pallas(x, W1, b1, W2, b2, W3, b3)


# speedup vs baseline: 2.6797x; 2.6797x over previous
import functools

import jax
import jax.numpy as jnp
from jax import lax
from jax.experimental import pallas as pl
from jax.experimental.pallas import tpu as pltpu
from jax.experimental.pallas import tpu_sc as plsc

NUM_NODES = 200
N = 6400
F = 32
PW = 128
E = 102400
T = 8
NSUB = 16
CHUNK = 128


def _scale_rows(rows_v, norm_v):
    def body(g, _):
        nrm = norm_v[pl.ds(g * 16, 16)]
        for l in range(16):
            s = nrm[l]
            e = g * 16 + l
            for k in range(F // 16):
                rows_v[e, pl.ds(k * 16, 16)] = (
                    rows_v[e, pl.ds(k * 16, 16)] * s)
        return 0

    lax.fori_loop(0, CHUNK // 16, body, 0)


def _edge_pass(x_hbm, src_hbm, dst_hbm, norm_hbm, accum, src_v, dst_v,
               norm_v, rows_v, sem, wid):
    epw = E // NSUB

    def body(i, _):
        base = wid * epw + i * CHUNK
        pltpu.sync_copy(src_hbm.at[pl.ds(base, CHUNK)], src_v)
        pltpu.sync_copy(dst_hbm.at[pl.ds(base, CHUNK)], dst_v)
        pltpu.sync_copy(norm_hbm.at[pl.ds(base, CHUNK)], norm_v)
        pltpu.async_copy(x_hbm.at[src_v], rows_v, sem).wait()
        _scale_rows(rows_v, norm_v)
        pltpu.sync_copy(rows_v, accum.at[dst_v], add=True)
        return 0

    lax.fori_loop(0, epw // CHUNK, body, 0)


def _make_prop_pair():
    rpw = N // NSUB
    zr = 80
    mesh = plsc.VectorSubcoreMesh(core_axis_name="c", subcore_axis_name="s",
                                  num_cores=1)

    @functools.partial(
        pl.kernel, mesh=mesh,
        out_type=[
            jax.ShapeDtypeStruct((N, PW), jnp.float32),
            jax.ShapeDtypeStruct((N, PW), jnp.float32),
        ],
        scratch_types=[
            pltpu.VMEM((CHUNK,), jnp.int32),
            pltpu.VMEM((CHUNK,), jnp.int32),
            pltpu.VMEM((CHUNK,), jnp.float32),
            pltpu.VMEM((CHUNK, PW), jnp.float32),
            pltpu.VMEM((zr, PW), jnp.float32),
            pltpu.VMEM_SHARED((N, PW), jnp.float32),
            pltpu.SemaphoreType.DMA,
        ],
    )
    def kfn(x_hbm, src_hbm, dst_hbm, norm_hbm, p1_hbm, p2_hbm,
            src_v, dst_v, norm_v, rows_v, zv, acc, sem):
        wid = lax.axis_index("s")

        def zero_buf(buf, rows):
            def body(i, _):
                for k in range(PW // 16):
                    buf[i, pl.ds(k * 16, 16)] = jnp.zeros((16,), jnp.float32)
                return 0

            lax.fori_loop(0, rows, body, 0)

        zero_buf(zv, zr)

        def zero_acc():
            for j in range(rpw // zr):
                pltpu.sync_copy(zv, acc.at[pl.ds(wid * rpw + j * zr, zr)])

        zero_acc()
        plsc.subcore_barrier()
        _edge_pass(x_hbm, src_hbm, dst_hbm, norm_hbm, acc, src_v, dst_v,
                   norm_v, rows_v, sem, wid)
        plsc.subcore_barrier()
        pltpu.sync_copy(acc.at[pl.ds(wid * rpw, rpw)],
                        p1_hbm.at[pl.ds(wid * rpw, rpw)])
        plsc.subcore_barrier()
        zero_acc()
        plsc.subcore_barrier()
        _edge_pass(p1_hbm, src_hbm, dst_hbm, norm_hbm, acc, src_v, dst_v,
                   norm_v, rows_v, sem, wid)
        plsc.subcore_barrier()
        pltpu.sync_copy(acc.at[pl.ds(wid * rpw, rpw)],
                        p2_hbm.at[pl.ds(wid * rpw, rpw)])

    return kfn


_prop_pair = _make_prop_pair()


def _tc_gx(xall, p1x, p2x, C0, C1, C2, bgx):
    def body(x_ref, p1_ref, p2_ref, c0, c1, c2, bg, out_ref):
        acc = jnp.dot(x_ref[:, :], c0[:, :],
                      preferred_element_type=jnp.float32)
        acc += jnp.dot(p1_ref[:, :F], c1[:, :],
                       preferred_element_type=jnp.float32)
        acc += jnp.dot(p2_ref[:, :F], c2[:, :],
                       preferred_element_type=jnp.float32)
        out_ref[:, :] = acc + bg[0, :]

    return pl.pallas_call(
        body,
        grid=(T,),
        in_specs=[
            pl.BlockSpec((N, F), lambda j: (j, 0)),
            pl.BlockSpec((N, PW), lambda j: (j, 0)),
            pl.BlockSpec((N, PW), lambda j: (j, 0)),
            pl.BlockSpec((F, 96), lambda j: (0, 0)),
            pl.BlockSpec((F, 96), lambda j: (0, 0)),
            pl.BlockSpec((F, 96), lambda j: (0, 0)),
            pl.BlockSpec((1, 96), lambda j: (0, 0)),
        ],
        out_specs=pl.BlockSpec((N, 96), lambda j: (j, 0)),
        out_shape=jax.ShapeDtypeStruct((T * N, 96), jnp.float32),
    )(xall, p1x, p2x, C0, C1, C2, bgx.reshape(1, 96))


def _tc_step1(gx):
    def body(g_ref, h_ref):
        g = g_ref[0]
        z = jax.nn.sigmoid(g[:, :32])
        ht = jnp.tanh(g[:, 64:])
        h_ref[:, :F] = jax.nn.relu((1.0 - z) * ht)
        h_ref[:, F:] = jnp.zeros((N, PW - F), jnp.float32)

    return pl.pallas_call(
        body,
        grid=(1,),
        in_specs=[pl.BlockSpec((1, N, 96), lambda i: (0, 0, 0))],
        out_specs=pl.BlockSpec((N, PW), lambda i: (0, 0)),
        out_shape=jax.ShapeDtypeStruct((N, PW), jnp.float32),
    )(gx)


def _tc_gate_zr(gx, t, H, p1h, p2h, A0, A1, A2):
    def body(g_ref, h_ref, p1_ref, p2_ref, a0, a1, a2, z_ref, rh_ref):
        h = h_ref[:, :F]
        acc = g_ref[0][:, :64]
        acc += jnp.dot(h, a0[:, :], preferred_element_type=jnp.float32)
        acc += jnp.dot(p1_ref[:, :F], a1[:, :],
                       preferred_element_type=jnp.float32)
        acc += jnp.dot(p2_ref[:, :F], a2[:, :],
                       preferred_element_type=jnp.float32)
        zr = jax.nn.sigmoid(acc)
        z_ref[:, :] = zr[:, :32]
        rh_ref[:, :F] = zr[:, 32:] * h
        rh_ref[:, F:] = jnp.zeros((N, PW - F), jnp.float32)

    return pl.pallas_call(
        body,
        grid=(1,),
        in_specs=[
            pl.BlockSpec((1, N, 96), lambda i: (t, 0, 0)),
            pl.BlockSpec((N, PW), lambda i: (0, 0)),
            pl.BlockSpec((N, PW), lambda i: (0, 0)),
            pl.BlockSpec((N, PW), lambda i: (0, 0)),
            pl.BlockSpec((F, 64), lambda i: (0, 0)),
            pl.BlockSpec((F, 64), lambda i: (0, 0)),
            pl.BlockSpec((F, 64), lambda i: (0, 0)),
        ],
        out_specs=[
            pl.BlockSpec((N, F), lambda i: (0, 0)),
            pl.BlockSpec((N, PW), lambda i: (0, 0)),
        ],
        out_shape=[
            jax.ShapeDtypeStruct((N, F), jnp.float32),
            jax.ShapeDtypeStruct((N, PW), jnp.float32),
        ],
    )(gx, H, p1h, p2h, A0, A1, A2)


def _tc_gate_h(gx, t, RH, p1r, p2r, Z, H, B0, B1, B2):
    def body(g_ref, rh_ref, p1_ref, p2_ref, z_ref, h_ref, b0, b1, b2,
             out_ref):
        acc = g_ref[0][:, 64:]
        acc += jnp.dot(rh_ref[:, :F], b0[:, :],
                       preferred_element_type=jnp.float32)
        acc += jnp.dot(p1_ref[:, :F], b1[:, :],
                       preferred_element_type=jnp.float32)
        acc += jnp.dot(p2_ref[:, :F], b2[:, :],
                       preferred_element_type=jnp.float32)
        ht = jnp.tanh(acc)
        z = z_ref[:, :]
        out_ref[:, :F] = jax.nn.relu(z * h_ref[:, :F] + (1.0 - z) * ht)
        out_ref[:, F:] = jnp.zeros((N, PW - F), jnp.float32)

    return pl.pallas_call(
        body,
        grid=(1,),
        in_specs=[
            pl.BlockSpec((1, N, 96), lambda i: (t, 0, 0)),
            pl.BlockSpec((N, PW), lambda i: (0, 0)),
            pl.BlockSpec((N, PW), lambda i: (0, 0)),
            pl.BlockSpec((N, PW), lambda i: (0, 0)),
            pl.BlockSpec((N, F), lambda i: (0, 0)),
            pl.BlockSpec((N, PW), lambda i: (0, 0)),
            pl.BlockSpec((F, F), lambda i: (0, 0)),
            pl.BlockSpec((F, F), lambda i: (0, 0)),
            pl.BlockSpec((F, F), lambda i: (0, 0)),
        ],
        out_specs=pl.BlockSpec((N, PW), lambda i: (0, 0)),
        out_shape=jax.ShapeDtypeStruct((N, PW), jnp.float32),
    )(gx, RH, p1r, p2r, Z, H, B0, B1, B2)


def _mlp_head_pallas(x, W1, b1, W2, b2, W3, b3):
    B, SIN = x.shape
    S1 = W1.shape[1]
    S2 = W2.shape[1]
    OUT = W3.shape[1]
    BLK = 640
    nb = S1 // BLK

    def body(x_ref, w1_ref, b1_ref, w2_ref, b2_ref, w3_ref, b3_ref, out_ref,
             h1_ref):
        j = pl.program_id(0)
        h1_ref[:, pl.ds(j * BLK, BLK)] = jnp.maximum(
            x_ref[:, :] @ w1_ref[:, :] + b1_ref[0, :], 0.0)

        @pl.when(j == nb - 1)
        def _():
            h2 = jnp.maximum(h1_ref[:, :] @ w2_ref[:, :] + b2_ref[0, :], 0.0)
            logits = h2 @ w3_ref[:, :] + b3_ref[0, :]
            m = jnp.max(logits, axis=-1, keepdims=True)
            e = jnp.exp(logits - m)
            out_ref[:, :] = e / jnp.sum(e, axis=-1, keepdims=True)

    return pl.pallas_call(
        body,
        grid=(nb,),
        in_specs=[
            pl.BlockSpec((B, SIN), lambda j: (0, 0)),
            pl.BlockSpec((SIN, BLK), lambda j: (0, j)),
            pl.BlockSpec((1, BLK), lambda j: (0, j)),
            pl.BlockSpec((S1, S2), lambda j: (0, 0)),
            pl.BlockSpec((1, S2), lambda j: (0, 0)),
            pl.BlockSpec((S2, OUT), lambda j: (0, 0)),
            pl.BlockSpec((1, OUT), lambda j: (0, 0)),
        ],
        out_specs=pl.BlockSpec((B, OUT), lambda j: (0, 0)),
        out_shape=jax.ShapeDtypeStruct((B, OUT), jnp.float32),
        scratch_shapes=[pltpu.VMEM((B, S1), jnp.float32)],
    )(x, W1, b1.reshape(1, -1), W2, b2.reshape(1, -1), W3, b3.reshape(1, -1))


def kernel(x_temporal, edge_index, edge_weight, batch, W_xz, b_xz, W_hz, b_hz,
           W_xr, b_xr, W_hr, b_hr, W_xh, b_xh, W_hh, b_hh, W1, b1, W2, b2,
           W3, b3):
    src, dst = edge_index[0], edge_index[1]
    deg = jnp.zeros((N,), x_temporal.dtype).at[dst].add(edge_weight)
    safe = jnp.where(deg > 0, deg, 1.0)
    dis = jnp.where(deg > 0, 1.0 / jnp.sqrt(safe), 0.0)
    norm = -dis[src] * edge_weight * dis[dst]

    C0 = jnp.concatenate([W_xz[0] - W_xz[2], W_xr[0] - W_xr[2],
                          W_xh[0] - W_xh[2]], axis=1)
    C1 = jnp.concatenate([W_xz[1], W_xr[1], W_xh[1]], axis=1)
    C2 = jnp.concatenate([2 * W_xz[2], 2 * W_xr[2], 2 * W_xh[2]], axis=1)
    bgx = jnp.concatenate([b_xz + b_hz, b_xr + b_hr, b_xh + b_hh])
    A0 = jnp.concatenate([W_hz[0] - W_hz[2], W_hr[0] - W_hr[2]], axis=1)
    A1 = jnp.concatenate([W_hz[1], W_hr[1]], axis=1)
    A2 = jnp.concatenate([2 * W_hz[2], 2 * W_hr[2]], axis=1)
    B0 = W_hh[0] - W_hh[2]
    B1 = W_hh[1]
    B2 = 2 * W_hh[2]

    xpad = jnp.pad(x_temporal, ((0, 0), (0, 0), (0, PW - F)))
    p1l, p2l = [], []
    for t in range(T):
        p1, p2 = _prop_pair(xpad[t], src, dst, norm)
        p1l.append(p1)
        p2l.append(p2)
    p1x = jnp.concatenate(p1l, axis=0)
    p2x = jnp.concatenate(p2l, axis=0)
    gx = _tc_gx(x_temporal.reshape(T * N, F), p1x, p2x, C0, C1, C2, bgx)
    gx = gx.reshape(T, N, 96)

    H = _tc_step1(gx)
    for t in range(1, T):
        p1h, p2h = _prop_pair(H, src, dst, norm)
        Z, RH = _tc_gate_zr(gx, t, H, p1h, p2h, A0, A1, A2)
        p1r, p2r = _prop_pair(RH, src, dst, norm)
        H = _tc_gate_h(gx, t, RH, p1r, p2r, Z, H, B0, B1, B2)

    bsz = -(-batch.shape[0] // NUM_NODES)
    x = H[:, :F].reshape(bsz, NUM_NODES * F)
    return _mlp_head_pallas(x, W1, b1, W2, b2, W3, b3)


# speedup vs baseline: 6.5829x; 2.4566x over previous
import functools

import jax
import jax.numpy as jnp
from jax import lax
from jax.experimental import pallas as pl
from jax.experimental.pallas import tpu as pltpu
from jax.experimental.pallas import tpu_sc as plsc

NUM_NODES = 200
N = 6400
F = 32
E = 102400
T = 8
NSUB = 16
CHUNK = 128
EPW = E // NSUB
NCH = EPW // CHUNK
RPW = N // NSUB


def _scale_rows(rows_v, norm_v, chunk_base):
    def body(g, _):
        nrm = norm_v[pl.ds(chunk_base + g * 16, 16)]
        for l in range(16):
            s = nrm[l]
            e = g * 16 + l
            for k in range(F // 16):
                rows_v[e, pl.ds(k * 16, 16)] = (
                    rows_v[e, pl.ds(k * 16, 16)] * s)
        return 0

    lax.fori_loop(0, CHUNK // 16, body, 0)


def _edge_pass(xs, src_v, dst_v, norm_v, accum, rows_v, sem):
    def body(i, _):
        pltpu.async_copy(xs.at[src_v.at[i]], rows_v, sem).wait()
        _scale_rows(rows_v, norm_v, i * CHUNK)
        pltpu.sync_copy(rows_v, accum.at[dst_v.at[i]], add=True)
        return 0

    lax.fori_loop(0, NCH, body, 0)


def _make_prop_pair():
    zr = 80
    mesh = plsc.VectorSubcoreMesh(core_axis_name="c", subcore_axis_name="s",
                                  num_cores=1)

    @functools.partial(
        pl.kernel, mesh=mesh,
        out_type=[
            jax.ShapeDtypeStruct((N, F), jnp.float32),
            jax.ShapeDtypeStruct((N, F), jnp.float32),
        ],
        scratch_types=[
            pltpu.VMEM((NCH, CHUNK), jnp.int32),
            pltpu.VMEM((NCH, CHUNK), jnp.int32),
            pltpu.VMEM((EPW,), jnp.float32),
            pltpu.VMEM((CHUNK, F), jnp.float32),
            pltpu.VMEM((zr, F), jnp.float32),
            pltpu.VMEM_SHARED((N, F), jnp.float32),
            pltpu.VMEM_SHARED((N, F), jnp.float32),
            pltpu.VMEM_SHARED((N, F), jnp.float32),
            pltpu.SemaphoreType.DMA,
        ],
    )
    def kfn(x_hbm, src_hbm, dst_hbm, norm_hbm, p1_hbm, p2_hbm,
            src_v, dst_v, norm_v, rows_v, zv, xs, acc1, acc2, sem):
        wid = lax.axis_index("s")
        pltpu.sync_copy(src_hbm.at[wid], src_v)
        pltpu.sync_copy(dst_hbm.at[wid], dst_v)
        pltpu.sync_copy(norm_hbm.at[wid], norm_v)
        rows = pl.ds(wid * RPW, RPW)
        pltpu.sync_copy(x_hbm.at[rows], xs.at[rows])

        def zero_buf(i, _):
            for k in range(F // 16):
                zv[i, pl.ds(k * 16, 16)] = jnp.zeros((16,), jnp.float32)
            return 0

        lax.fori_loop(0, zr, zero_buf, 0)
        for j in range(RPW // zr):
            sl = pl.ds(wid * RPW + j * zr, zr)
            pltpu.sync_copy(zv, acc1.at[sl])
            pltpu.sync_copy(zv, acc2.at[sl])
        plsc.subcore_barrier()
        _edge_pass(xs, src_v, dst_v, norm_v, acc1, rows_v, sem)
        plsc.subcore_barrier()
        pltpu.sync_copy(acc1.at[rows], p1_hbm.at[rows])
        _edge_pass(acc1, src_v, dst_v, norm_v, acc2, rows_v, sem)
        plsc.subcore_barrier()
        pltpu.sync_copy(acc2.at[rows], p2_hbm.at[rows])

    return kfn


_prop_pair_raw = _make_prop_pair()


def _prop_pair(x, src3, dst3, norm2):
    return _prop_pair_raw(x, src3, dst3, norm2)


def _tc_gx(xall, p1x, p2x, C0, C1, C2, bgx):
    def body(x_ref, p1_ref, p2_ref, c0, c1, c2, bg, out_ref):
        acc = jnp.dot(x_ref[:, :], c0[:, :],
                      preferred_element_type=jnp.float32)
        acc += jnp.dot(p1_ref[:, :], c1[:, :],
                       preferred_element_type=jnp.float32)
        acc += jnp.dot(p2_ref[:, :], c2[:, :],
                       preferred_element_type=jnp.float32)
        out_ref[:, :] = acc + bg[0, :]

    return pl.pallas_call(
        body,
        grid=(T,),
        in_specs=[
            pl.BlockSpec((N, F), lambda j: (j, 0)),
            pl.BlockSpec((N, F), lambda j: (j, 0)),
            pl.BlockSpec((N, F), lambda j: (j, 0)),
            pl.BlockSpec((F, 96), lambda j: (0, 0)),
            pl.BlockSpec((F, 96), lambda j: (0, 0)),
            pl.BlockSpec((F, 96), lambda j: (0, 0)),
            pl.BlockSpec((1, 96), lambda j: (0, 0)),
        ],
        out_specs=pl.BlockSpec((N, 96), lambda j: (j, 0)),
        out_shape=jax.ShapeDtypeStruct((T * N, 96), jnp.float32),
    )(xall, p1x, p2x, C0, C1, C2, bgx.reshape(1, 96))


def _tc_step1(gx):
    def body(g_ref, h_ref):
        g = g_ref[0]
        z = jax.nn.sigmoid(g[:, :32])
        ht = jnp.tanh(g[:, 64:])
        h_ref[:, :] = jax.nn.relu((1.0 - z) * ht)

    return pl.pallas_call(
        body,
        grid=(1,),
        in_specs=[pl.BlockSpec((1, N, 96), lambda i: (0, 0, 0))],
        out_specs=pl.BlockSpec((N, F), lambda i: (0, 0)),
        out_shape=jax.ShapeDtypeStruct((N, F), jnp.float32),
    )(gx)


def _tc_gate_zr(gx, t, H, p1h, p2h, A0, A1, A2):
    def body(g_ref, h_ref, p1_ref, p2_ref, a0, a1, a2, z_ref, rh_ref):
        h = h_ref[:, :]
        acc = g_ref[0][:, :64]
        acc += jnp.dot(h, a0[:, :], preferred_element_type=jnp.float32)
        acc += jnp.dot(p1_ref[:, :], a1[:, :],
                       preferred_element_type=jnp.float32)
        acc += jnp.dot(p2_ref[:, :], a2[:, :],
                       preferred_element_type=jnp.float32)
        zr = jax.nn.sigmoid(acc)
        z_ref[:, :] = zr[:, :32]
        rh_ref[:, :] = zr[:, 32:] * h

    return pl.pallas_call(
        body,
        grid=(1,),
        in_specs=[
            pl.BlockSpec((1, N, 96), lambda i: (t, 0, 0)),
            pl.BlockSpec((N, F), lambda i: (0, 0)),
            pl.BlockSpec((N, F), lambda i: (0, 0)),
            pl.BlockSpec((N, F), lambda i: (0, 0)),
            pl.BlockSpec((F, 64), lambda i: (0, 0)),
            pl.BlockSpec((F, 64), lambda i: (0, 0)),
            pl.BlockSpec((F, 64), lambda i: (0, 0)),
        ],
        out_specs=[
            pl.BlockSpec((N, F), lambda i: (0, 0)),
            pl.BlockSpec((N, F), lambda i: (0, 0)),
        ],
        out_shape=[
            jax.ShapeDtypeStruct((N, F), jnp.float32),
            jax.ShapeDtypeStruct((N, F), jnp.float32),
        ],
    )(gx, H, p1h, p2h, A0, A1, A2)


def _tc_gate_h(gx, t, RH, p1r, p2r, Z, H, B0, B1, B2):
    def body(g_ref, rh_ref, p1_ref, p2_ref, z_ref, h_ref, b0, b1, b2,
             out_ref):
        acc = g_ref[0][:, 64:]
        acc += jnp.dot(rh_ref[:, :], b0[:, :],
                       preferred_element_type=jnp.float32)
        acc += jnp.dot(p1_ref[:, :], b1[:, :],
                       preferred_element_type=jnp.float32)
        acc += jnp.dot(p2_ref[:, :], b2[:, :],
                       preferred_element_type=jnp.float32)
        ht = jnp.tanh(acc)
        z = z_ref[:, :]
        out_ref[:, :] = jax.nn.relu(z * h_ref[:, :] + (1.0 - z) * ht)

    return pl.pallas_call(
        body,
        grid=(1,),
        in_specs=[
            pl.BlockSpec((1, N, 96), lambda i: (t, 0, 0)),
            pl.BlockSpec((N, F), lambda i: (0, 0)),
            pl.BlockSpec((N, F), lambda i: (0, 0)),
            pl.BlockSpec((N, F), lambda i: (0, 0)),
            pl.BlockSpec((N, F), lambda i: (0, 0)),
            pl.BlockSpec((N, F), lambda i: (0, 0)),
            pl.BlockSpec((F, F), lambda i: (0, 0)),
            pl.BlockSpec((F, F), lambda i: (0, 0)),
            pl.BlockSpec((F, F), lambda i: (0, 0)),
        ],
        out_specs=pl.BlockSpec((N, F), lambda i: (0, 0)),
        out_shape=jax.ShapeDtypeStruct((N, F), jnp.float32),
    )(gx, RH, p1r, p2r, Z, H, B0, B1, B2)


def _mlp_head_pallas(x, W1, b1, W2, b2, W3, b3):
    B, SIN = x.shape
    S1 = W1.shape[1]
    S2 = W2.shape[1]
    OUT = W3.shape[1]
    BLK = 640
    nb = S1 // BLK

    def body(x_ref, w1_ref, b1_ref, w2_ref, b2_ref, w3_ref, b3_ref, out_ref,
             h1_ref):
        j = pl.program_id(0)
        h1_ref[:, pl.ds(j * BLK, BLK)] = jnp.maximum(
            x_ref[:, :] @ w1_ref[:, :] + b1_ref[0, :], 0.0)

        @pl.when(j == nb - 1)
        def _():
            h2 = jnp.maximum(h1_ref[:, :] @ w2_ref[:, :] + b2_ref[0, :], 0.0)
            logits = h2 @ w3_ref[:, :] + b3_ref[0, :]
            m = jnp.max(logits, axis=-1, keepdims=True)
            e = jnp.exp(logits - m)
            out_ref[:, :] = e / jnp.sum(e, axis=-1, keepdims=True)

    return pl.pallas_call(
        body,
        grid=(nb,),
        in_specs=[
            pl.BlockSpec((B, SIN), lambda j: (0, 0)),
            pl.BlockSpec((SIN, BLK), lambda j: (0, j)),
            pl.BlockSpec((1, BLK), lambda j: (0, j)),
            pl.BlockSpec((S1, S2), lambda j: (0, 0)),
            pl.BlockSpec((1, S2), lambda j: (0, 0)),
            pl.BlockSpec((S2, OUT), lambda j: (0, 0)),
            pl.BlockSpec((1, OUT), lambda j: (0, 0)),
        ],
        out_specs=pl.BlockSpec((B, OUT), lambda j: (0, 0)),
        out_shape=jax.ShapeDtypeStruct((B, OUT), jnp.float32),
        scratch_shapes=[pltpu.VMEM((B, S1), jnp.float32)],
    )(x, W1, b1.reshape(1, -1), W2, b2.reshape(1, -1), W3, b3.reshape(1, -1))


def kernel(x_temporal, edge_index, edge_weight, batch, W_xz, b_xz, W_hz, b_hz,
           W_xr, b_xr, W_hr, b_hr, W_xh, b_xh, W_hh, b_hh, W1, b1, W2, b2,
           W3, b3):
    src, dst = edge_index[0], edge_index[1]
    deg = jnp.zeros((N,), x_temporal.dtype).at[dst].add(edge_weight)
    safe = jnp.where(deg > 0, deg, 1.0)
    dis = jnp.where(deg > 0, 1.0 / jnp.sqrt(safe), 0.0)
    norm = -dis[src] * edge_weight * dis[dst]

    src3 = src.reshape(NSUB, NCH, CHUNK)
    dst3 = dst.reshape(NSUB, NCH, CHUNK)
    norm2 = norm.reshape(NSUB, EPW)

    C0 = jnp.concatenate([W_xz[0] - W_xz[2], W_xr[0] - W_xr[2],
                          W_xh[0] - W_xh[2]], axis=1)
    C1 = jnp.concatenate([W_xz[1], W_xr[1], W_xh[1]], axis=1)
    C2 = jnp.concatenate([2 * W_xz[2], 2 * W_xr[2], 2 * W_xh[2]], axis=1)
    bgx = jnp.concatenate([b_xz + b_hz, b_xr + b_hr, b_xh + b_hh])
    A0 = jnp.concatenate([W_hz[0] - W_hz[2], W_hr[0] - W_hr[2]], axis=1)
    A1 = jnp.concatenate([W_hz[1], W_hr[1]], axis=1)
    A2 = jnp.concatenate([2 * W_hz[2], 2 * W_hr[2]], axis=1)
    B0 = W_hh[0] - W_hh[2]
    B1 = W_hh[1]
    B2 = 2 * W_hh[2]

    p1l, p2l = [], []
    for t in range(T):
        p1, p2 = _prop_pair(x_temporal[t], src3, dst3, norm2)
        p1l.append(p1)
        p2l.append(p2)
    p1x = jnp.concatenate(p1l, axis=0)
    p2x = jnp.concatenate(p2l, axis=0)
    gx = _tc_gx(x_temporal.reshape(T * N, F), p1x, p2x, C0, C1, C2, bgx)
    gx = gx.reshape(T, N, 96)

    H = _tc_step1(gx)
    for t in range(1, T):
        p1h, p2h = _prop_pair(H, src3, dst3, norm2)
        Z, RH = _tc_gate_zr(gx, t, H, p1h, p2h, A0, A1, A2)
        p1r, p2r = _prop_pair(RH, src3, dst3, norm2)
        H = _tc_gate_h(gx, t, RH, p1r, p2r, Z, H, B0, B1, B2)

    bsz = -(-batch.shape[0] // NUM_NODES)
    x = H.reshape(bsz, NUM_NODES * F)
    return _mlp_head_pallas(x, W1, b1, W2, b2, W3, b3)


# speedup vs baseline: 6.8389x; 1.0389x over previous
import functools

import jax
import jax.numpy as jnp
from jax import lax
from jax.experimental import pallas as pl
from jax.experimental.pallas import tpu as pltpu
from jax.experimental.pallas import tpu_sc as plsc

NUM_NODES = 200
N = 6400
F = 32
E = 102400
T = 8
NSUB = 16
CHUNK = 128
EPW = E // NSUB
NCH = EPW // CHUNK
RPW = N // NSUB


def _scale_rows(rows_v, norm_v, chunk_base):
    def body(g, _):
        nrm = norm_v[pl.ds(chunk_base + g * 16, 16)]
        for l in range(16):
            s = nrm[l]
            e = g * 16 + l
            for k in range(F // 16):
                rows_v[e, pl.ds(k * 16, 16)] = (
                    rows_v[e, pl.ds(k * 16, 16)] * s)
        return 0

    lax.fori_loop(0, CHUNK // 16, body, 0)


def _edge_pass(xs, src_v, dst_v, norm_v, accum, rows_v, sem):
    def body(i, _):
        pltpu.async_copy(xs.at[src_v.at[i]], rows_v, sem).wait()
        _scale_rows(rows_v, norm_v, i * CHUNK)
        pltpu.sync_copy(rows_v, accum.at[dst_v.at[i]], add=True)
        return 0

    lax.fori_loop(0, NCH, body, 0)


def _make_prop_pair():
    zr = 80
    mesh = plsc.VectorSubcoreMesh(core_axis_name="c", subcore_axis_name="s",
                                  num_cores=1)

    @functools.partial(
        pl.kernel, mesh=mesh,
        out_type=[
            jax.ShapeDtypeStruct((N, F), jnp.float32),
            jax.ShapeDtypeStruct((N, F), jnp.float32),
        ],
        scratch_types=[
            pltpu.VMEM((NCH, CHUNK), jnp.int32),
            pltpu.VMEM((NCH, CHUNK), jnp.int32),
            pltpu.VMEM((EPW,), jnp.float32),
            pltpu.VMEM((CHUNK, F), jnp.float32),
            pltpu.VMEM((zr, F), jnp.float32),
            pltpu.VMEM_SHARED((N, F), jnp.float32),
            pltpu.VMEM_SHARED((N, F), jnp.float32),
            pltpu.VMEM_SHARED((N, F), jnp.float32),
            pltpu.SemaphoreType.DMA,
        ],
    )
    def kfn(x_hbm, src_hbm, dst_hbm, norm_hbm, p1_hbm, p2_hbm,
            src_v, dst_v, norm_v, rows_v, zv, xs, acc1, acc2, sem):
        wid = lax.axis_index("s")
        pltpu.sync_copy(src_hbm.at[wid], src_v)
        pltpu.sync_copy(dst_hbm.at[wid], dst_v)
        pltpu.sync_copy(norm_hbm.at[wid], norm_v)
        rows = pl.ds(wid * RPW, RPW)
        pltpu.sync_copy(x_hbm.at[rows], xs.at[rows])

        def zero_buf(i, _):
            for k in range(F // 16):
                zv[i, pl.ds(k * 16, 16)] = jnp.zeros((16,), jnp.float32)
            return 0

        lax.fori_loop(0, zr, zero_buf, 0)
        for j in range(RPW // zr):
            sl = pl.ds(wid * RPW + j * zr, zr)
            pltpu.sync_copy(zv, acc1.at[sl])
            pltpu.sync_copy(zv, acc2.at[sl])
        plsc.subcore_barrier()
        _edge_pass(xs, src_v, dst_v, norm_v, acc1, rows_v, sem)
        plsc.subcore_barrier()
        pltpu.sync_copy(acc1.at[rows], p1_hbm.at[rows])
        _edge_pass(acc1, src_v, dst_v, norm_v, acc2, rows_v, sem)
        plsc.subcore_barrier()
        pltpu.sync_copy(acc2.at[rows], p2_hbm.at[rows])

    return kfn


_prop_pair_raw = _make_prop_pair()


def _prop_pair(x, src3, dst3, norm2):
    return _prop_pair_raw(x, src3, dst3, norm2)


def _make_prop_pair_all_t():
    zr = 80
    mesh = plsc.VectorSubcoreMesh(core_axis_name="c", subcore_axis_name="s",
                                  num_cores=1)

    @functools.partial(
        pl.kernel, mesh=mesh,
        out_type=[
            jax.ShapeDtypeStruct((T, N, F), jnp.float32),
            jax.ShapeDtypeStruct((T, N, F), jnp.float32),
        ],
        scratch_types=[
            pltpu.VMEM((NCH, CHUNK), jnp.int32),
            pltpu.VMEM((NCH, CHUNK), jnp.int32),
            pltpu.VMEM((EPW,), jnp.float32),
            pltpu.VMEM((CHUNK, F), jnp.float32),
            pltpu.VMEM((zr, F), jnp.float32),
            pltpu.VMEM_SHARED((N, F), jnp.float32),
            pltpu.VMEM_SHARED((N, F), jnp.float32),
            pltpu.VMEM_SHARED((N, F), jnp.float32),
            pltpu.SemaphoreType.DMA,
        ],
    )
    def kfn(x_hbm, src_hbm, dst_hbm, norm_hbm, p1_hbm, p2_hbm,
            src_v, dst_v, norm_v, rows_v, zv, xs, acc1, acc2, sem):
        wid = lax.axis_index("s")
        pltpu.sync_copy(src_hbm.at[wid], src_v)
        pltpu.sync_copy(dst_hbm.at[wid], dst_v)
        pltpu.sync_copy(norm_hbm.at[wid], norm_v)
        rows = pl.ds(wid * RPW, RPW)

        def zero_buf(i, _):
            for k in range(F // 16):
                zv[i, pl.ds(k * 16, 16)] = jnp.zeros((16,), jnp.float32)
            return 0

        lax.fori_loop(0, zr, zero_buf, 0)

        def tbody(t, _):
            for j in range(RPW // zr):
                sl = pl.ds(wid * RPW + j * zr, zr)
                pltpu.sync_copy(zv, acc1.at[sl])
                pltpu.sync_copy(zv, acc2.at[sl])
            pltpu.sync_copy(x_hbm.at[t, rows], xs.at[rows])
            plsc.subcore_barrier()
            _edge_pass(xs, src_v, dst_v, norm_v, acc1, rows_v, sem)
            plsc.subcore_barrier()
            pltpu.sync_copy(acc1.at[rows], p1_hbm.at[t, rows])
            _edge_pass(acc1, src_v, dst_v, norm_v, acc2, rows_v, sem)
            plsc.subcore_barrier()
            pltpu.sync_copy(acc2.at[rows], p2_hbm.at[t, rows])
            return 0

        lax.fori_loop(0, T, tbody, 0)

    return kfn


_prop_pair_all_t = _make_prop_pair_all_t()


def _tc_gx(xall, p1x, p2x, C0, C1, C2, bgx):
    def body(x_ref, p1_ref, p2_ref, c0, c1, c2, bg, out_ref):
        acc = jnp.dot(x_ref[:, :], c0[:, :],
                      preferred_element_type=jnp.float32)
        acc += jnp.dot(p1_ref[:, :], c1[:, :],
                       preferred_element_type=jnp.float32)
        acc += jnp.dot(p2_ref[:, :], c2[:, :],
                       preferred_element_type=jnp.float32)
        out_ref[:, :] = acc + bg[0, :]

    return pl.pallas_call(
        body,
        grid=(T,),
        in_specs=[
            pl.BlockSpec((N, F), lambda j: (j, 0)),
            pl.BlockSpec((N, F), lambda j: (j, 0)),
            pl.BlockSpec((N, F), lambda j: (j, 0)),
            pl.BlockSpec((F, 96), lambda j: (0, 0)),
            pl.BlockSpec((F, 96), lambda j: (0, 0)),
            pl.BlockSpec((F, 96), lambda j: (0, 0)),
            pl.BlockSpec((1, 96), lambda j: (0, 0)),
        ],
        out_specs=pl.BlockSpec((N, 96), lambda j: (j, 0)),
        out_shape=jax.ShapeDtypeStruct((T * N, 96), jnp.float32),
    )(xall, p1x, p2x, C0, C1, C2, bgx.reshape(1, 96))


def _tc_step1(gx):
    def body(g_ref, h_ref):
        g = g_ref[0]
        z = jax.nn.sigmoid(g[:, :32])
        ht = jnp.tanh(g[:, 64:])
        h_ref[:, :] = jax.nn.relu((1.0 - z) * ht)

    return pl.pallas_call(
        body,
        grid=(1,),
        in_specs=[pl.BlockSpec((1, N, 96), lambda i: (0, 0, 0))],
        out_specs=pl.BlockSpec((N, F), lambda i: (0, 0)),
        out_shape=jax.ShapeDtypeStruct((N, F), jnp.float32),
    )(gx)


def _tc_gate_zr(gx, t, H, p1h, p2h, A0, A1, A2):
    def body(g_ref, h_ref, p1_ref, p2_ref, a0, a1, a2, z_ref, rh_ref):
        h = h_ref[:, :]
        acc = g_ref[0][:, :64]
        acc += jnp.dot(h, a0[:, :], preferred_element_type=jnp.float32)
        acc += jnp.dot(p1_ref[:, :], a1[:, :],
                       preferred_element_type=jnp.float32)
        acc += jnp.dot(p2_ref[:, :], a2[:, :],
                       preferred_element_type=jnp.float32)
        zr = jax.nn.sigmoid(acc)
        z_ref[:, :] = zr[:, :32]
        rh_ref[:, :] = zr[:, 32:] * h

    return pl.pallas_call(
        body,
        grid=(1,),
        in_specs=[
            pl.BlockSpec((1, N, 96), lambda i: (t, 0, 0)),
            pl.BlockSpec((N, F), lambda i: (0, 0)),
            pl.BlockSpec((N, F), lambda i: (0, 0)),
            pl.BlockSpec((N, F), lambda i: (0, 0)),
            pl.BlockSpec((F, 64), lambda i: (0, 0)),
            pl.BlockSpec((F, 64), lambda i: (0, 0)),
            pl.BlockSpec((F, 64), lambda i: (0, 0)),
        ],
        out_specs=[
            pl.BlockSpec((N, F), lambda i: (0, 0)),
            pl.BlockSpec((N, F), lambda i: (0, 0)),
        ],
        out_shape=[
            jax.ShapeDtypeStruct((N, F), jnp.float32),
            jax.ShapeDtypeStruct((N, F), jnp.float32),
        ],
    )(gx, H, p1h, p2h, A0, A1, A2)


def _tc_gate_h(gx, t, RH, p1r, p2r, Z, H, B0, B1, B2):
    def body(g_ref, rh_ref, p1_ref, p2_ref, z_ref, h_ref, b0, b1, b2,
             out_ref):
        acc = g_ref[0][:, 64:]
        acc += jnp.dot(rh_ref[:, :], b0[:, :],
                       preferred_element_type=jnp.float32)
        acc += jnp.dot(p1_ref[:, :], b1[:, :],
                       preferred_element_type=jnp.float32)
        acc += jnp.dot(p2_ref[:, :], b2[:, :],
                       preferred_element_type=jnp.float32)
        ht = jnp.tanh(acc)
        z = z_ref[:, :]
        out_ref[:, :] = jax.nn.relu(z * h_ref[:, :] + (1.0 - z) * ht)

    return pl.pallas_call(
        body,
        grid=(1,),
        in_specs=[
            pl.BlockSpec((1, N, 96), lambda i: (t, 0, 0)),
            pl.BlockSpec((N, F), lambda i: (0, 0)),
            pl.BlockSpec((N, F), lambda i: (0, 0)),
            pl.BlockSpec((N, F), lambda i: (0, 0)),
            pl.BlockSpec((N, F), lambda i: (0, 0)),
            pl.BlockSpec((N, F), lambda i: (0, 0)),
            pl.BlockSpec((F, F), lambda i: (0, 0)),
            pl.BlockSpec((F, F), lambda i: (0, 0)),
            pl.BlockSpec((F, F), lambda i: (0, 0)),
        ],
        out_specs=pl.BlockSpec((N, F), lambda i: (0, 0)),
        out_shape=jax.ShapeDtypeStruct((N, F), jnp.float32),
    )(gx, RH, p1r, p2r, Z, H, B0, B1, B2)


def _mlp_head_pallas(x, W1, b1, W2, b2, W3, b3):
    B, SIN = x.shape
    S1 = W1.shape[1]
    S2 = W2.shape[1]
    OUT = W3.shape[1]
    BLK = 640
    nb = S1 // BLK

    def body(x_ref, w1_ref, b1_ref, w2_ref, b2_ref, w3_ref, b3_ref, out_ref,
             h1_ref):
        j = pl.program_id(0)
        h1_ref[:, pl.ds(j * BLK, BLK)] = jnp.maximum(
            x_ref[:, :] @ w1_ref[:, :] + b1_ref[0, :], 0.0)

        @pl.when(j == nb - 1)
        def _():
            h2 = jnp.maximum(h1_ref[:, :] @ w2_ref[:, :] + b2_ref[0, :], 0.0)
            logits = h2 @ w3_ref[:, :] + b3_ref[0, :]
            m = jnp.max(logits, axis=-1, keepdims=True)
            e = jnp.exp(logits - m)
            out_ref[:, :] = e / jnp.sum(e, axis=-1, keepdims=True)

    return pl.pallas_call(
        body,
        grid=(nb,),
        in_specs=[
            pl.BlockSpec((B, SIN), lambda j: (0, 0)),
            pl.BlockSpec((SIN, BLK), lambda j: (0, j)),
            pl.BlockSpec((1, BLK), lambda j: (0, j)),
            pl.BlockSpec((S1, S2), lambda j: (0, 0)),
            pl.BlockSpec((1, S2), lambda j: (0, 0)),
            pl.BlockSpec((S2, OUT), lambda j: (0, 0)),
            pl.BlockSpec((1, OUT), lambda j: (0, 0)),
        ],
        out_specs=pl.BlockSpec((B, OUT), lambda j: (0, 0)),
        out_shape=jax.ShapeDtypeStruct((B, OUT), jnp.float32),
        scratch_shapes=[pltpu.VMEM((B, S1), jnp.float32)],
    )(x, W1, b1.reshape(1, -1), W2, b2.reshape(1, -1), W3, b3.reshape(1, -1))


def kernel(x_temporal, edge_index, edge_weight, batch, W_xz, b_xz, W_hz, b_hz,
           W_xr, b_xr, W_hr, b_hr, W_xh, b_xh, W_hh, b_hh, W1, b1, W2, b2,
           W3, b3):
    src, dst = edge_index[0], edge_index[1]
    deg = jnp.zeros((N,), x_temporal.dtype).at[dst].add(edge_weight)
    safe = jnp.where(deg > 0, deg, 1.0)
    dis = jnp.where(deg > 0, 1.0 / jnp.sqrt(safe), 0.0)
    norm = -dis[src] * edge_weight * dis[dst]

    src3 = src.reshape(NSUB, NCH, CHUNK)
    dst3 = dst.reshape(NSUB, NCH, CHUNK)
    norm2 = norm.reshape(NSUB, EPW)

    C0 = jnp.concatenate([W_xz[0] - W_xz[2], W_xr[0] - W_xr[2],
                          W_xh[0] - W_xh[2]], axis=1)
    C1 = jnp.concatenate([W_xz[1], W_xr[1], W_xh[1]], axis=1)
    C2 = jnp.concatenate([2 * W_xz[2], 2 * W_xr[2], 2 * W_xh[2]], axis=1)
    bgx = jnp.concatenate([b_xz + b_hz, b_xr + b_hr, b_xh + b_hh])
    A0 = jnp.concatenate([W_hz[0] - W_hz[2], W_hr[0] - W_hr[2]], axis=1)
    A1 = jnp.concatenate([W_hz[1], W_hr[1]], axis=1)
    A2 = jnp.concatenate([2 * W_hz[2], 2 * W_hr[2]], axis=1)
    B0 = W_hh[0] - W_hh[2]
    B1 = W_hh[1]
    B2 = 2 * W_hh[2]

    p1x3, p2x3 = _prop_pair_all_t(x_temporal, src3, dst3, norm2)
    p1x = p1x3.reshape(T * N, F)
    p2x = p2x3.reshape(T * N, F)
    gx = _tc_gx(x_temporal.reshape(T * N, F), p1x, p2x, C0, C1, C2, bgx)
    gx = gx.reshape(T, N, 96)

    H = _tc_step1(gx)
    for t in range(1, T):
        p1h, p2h = _prop_pair(H, src3, dst3, norm2)
        Z, RH = _tc_gate_zr(gx, t, H, p1h, p2h, A0, A1, A2)
        p1r, p2r = _prop_pair(RH, src3, dst3, norm2)
        H = _tc_gate_h(gx, t, RH, p1r, p2r, Z, H, B0, B1, B2)

    bsz = -(-batch.shape[0] // NUM_NODES)
    x = H.reshape(bsz, NUM_NODES * F)
    return _mlp_head_pallas(x, W1, b1, W2, b2, W3, b3)
